# speedup vs baseline: 6.3263x; 6.3263x over previous
import functools

import jax
import jax.numpy as jnp
from jax import lax
from jax.experimental import pallas as pl
from jax.experimental.pallas import tpu as pltpu
from jax.experimental.pallas import tpu_sc as plsc

VOCAB = 100000
EMBED_DIM = 128
FLAT = 4096 * 200
NUM_WORKERS = 32
PER_W = FLAT // NUM_WORKERS
CHUNK = 128
NCHUNK = PER_W // CHUNK


def _gather_kernel(idx_hbm, table_hbm, out_hbm, idx_v, rows_v, sem_in, sem_out):
    wid = lax.axis_index("s") * 2 + lax.axis_index("c")
    base = wid * PER_W

    pltpu.sync_copy(idx_hbm.at[pl.ds(base, PER_W)], idx_v)

    @pl.loop(0, NCHUNK)
    def _chunk(g):
        off = g * CHUNK
        pltpu.async_copy(
            table_hbm.at[idx_v.at[pl.ds(off, CHUNK)]],
            rows_v,
            sem_in,
        ).wait()
        pltpu.async_copy(
            rows_v,
            out_hbm.at[pl.ds(base + off, CHUNK)],
            sem_out,
        ).wait()


@jax.jit
def _embed(idx_flat, table):
    mesh = plsc.VectorSubcoreMesh(core_axis_name="c", subcore_axis_name="s")
    f = pl.kernel(
        _gather_kernel,
        out_type=jax.ShapeDtypeStruct((FLAT, EMBED_DIM), jnp.float32),
        mesh=mesh,
        scratch_types=[
            pltpu.VMEM((PER_W,), jnp.int32),
            pltpu.VMEM((CHUNK, EMBED_DIM), jnp.float32),
            pltpu.SemaphoreType.DMA,
            pltpu.SemaphoreType.DMA,
        ],
    )
    return f(idx_flat, table)


def kernel(lang_x, table):
    idx_flat = lang_x.reshape(FLAT).astype(jnp.int32)
    out = _embed(idx_flat, table)
    return out.reshape(lang_x.shape[0], lang_x.shape[1], EMBED_DIM)


# speedup vs baseline: 9.1825x; 1.4515x over previous
import functools

import jax
import jax.numpy as jnp
from jax import lax
from jax.experimental import pallas as pl
from jax.experimental.pallas import tpu as pltpu
from jax.experimental.pallas import tpu_sc as plsc

VOCAB = 100000
EMBED_DIM = 128
FLAT = 4096 * 200
NUM_WORKERS = 32
PER_W = FLAT // NUM_WORKERS
CHUNK = 128
NCHUNK = PER_W // CHUNK


NBUF = 5
LOOKAHEAD = 3


def _gather_kernel(idx_hbm, table_hbm, out_hbm, idx_v, rows_v, *sems):
    sem_in = sems[:NBUF]
    sem_out = sems[NBUF:]
    wid = lax.axis_index("s") * 2 + lax.axis_index("c")
    base = wid * PER_W

    pltpu.sync_copy(idx_hbm.at[pl.ds(base, PER_W)], idx_v)

    def issue_gather(g, b):
        pltpu.async_copy(
            table_hbm.at[idx_v.at[pl.ds(g * CHUNK, CHUNK)]],
            rows_v.at[b],
            sem_in[b],
        )

    def wait_gather(g, b):
        pltpu.make_async_copy(
            table_hbm.at[idx_v.at[pl.ds(g * CHUNK, CHUNK)]],
            rows_v.at[b],
            sem_in[b],
        ).wait()

    def issue_write(g, b):
        pltpu.async_copy(
            rows_v.at[b],
            out_hbm.at[pl.ds(base + g * CHUNK, CHUNK)],
            sem_out[b],
        )

    def wait_write(g, b):
        pltpu.make_async_copy(
            rows_v.at[b],
            out_hbm.at[pl.ds(base + g * CHUNK, CHUNK)],
            sem_out[b],
        ).wait()

    def step(g, b, write_wait, issue):
        wait_gather(g, b)
        issue_write(g, b)
        bb = (b + LOOKAHEAD) % NBUF
        if issue:
            if write_wait:
                wait_write(g + LOOKAHEAD - NBUF, bb)
            issue_gather(g + LOOKAHEAD, bb)

    for b in range(LOOKAHEAD):
        issue_gather(b, b)

    for b in range(NBUF):
        step(b, b, write_wait=(b + LOOKAHEAD - NBUF) >= 0, issue=True)

    @pl.loop(NBUF, NCHUNK - NBUF, step=NBUF)
    def _block(g0):
        for b in range(NBUF):
            step(g0 + b, b, write_wait=True, issue=True)

    for b in range(NBUF):
        g = NCHUNK - NBUF + b
        step(g, b, write_wait=True, issue=(g + LOOKAHEAD) < NCHUNK)

    for g in range(NCHUNK - NBUF, NCHUNK):
        wait_write(g, g % NBUF)


@jax.jit
def _embed(idx_flat, table):
    mesh = plsc.VectorSubcoreMesh(core_axis_name="c", subcore_axis_name="s")
    f = pl.kernel(
        _gather_kernel,
        out_type=jax.ShapeDtypeStruct((FLAT, EMBED_DIM), jnp.float32),
        mesh=mesh,
        scratch_types=(
            [
                pltpu.VMEM((PER_W,), jnp.int32),
                pltpu.VMEM((NBUF, CHUNK, EMBED_DIM), jnp.float32),
            ]
            + [pltpu.SemaphoreType.DMA] * (2 * NBUF)
        ),
    )
    return f(idx_flat, table)


def kernel(lang_x, table):
    idx_flat = lang_x.reshape(FLAT).astype(jnp.int32)
    out = _embed(idx_flat, table)
    return out.reshape(lang_x.shape[0], lang_x.shape[1], EMBED_DIM)


# speedup vs baseline: 9.1930x; 1.0011x over previous
import functools

import jax
import jax.numpy as jnp
from jax import lax
from jax.experimental import pallas as pl
from jax.experimental.pallas import tpu as pltpu
from jax.experimental.pallas import tpu_sc as plsc

VOCAB = 100000
EMBED_DIM = 128
FLAT = 4096 * 200
NUM_WORKERS = 32
PER_W = FLAT // NUM_WORKERS
CHUNK = 128
NCHUNK = PER_W // CHUNK


NBUF = 5
LOOKAHEAD = 4


def _gather_kernel(idx_hbm, table_hbm, out_hbm, idx_v, rows_v, *sems):
    sem_in = sems[:NBUF]
    sem_out = sems[NBUF:]
    wid = lax.axis_index("s") * 2 + lax.axis_index("c")
    base = wid * PER_W

    pltpu.sync_copy(idx_hbm.at[pl.ds(base, PER_W)], idx_v)

    def issue_gather(g, b):
        pltpu.async_copy(
            table_hbm.at[idx_v.at[pl.ds(g * CHUNK, CHUNK)]],
            rows_v.at[b],
            sem_in[b],
        )

    def wait_gather(g, b):
        pltpu.make_async_copy(
            table_hbm.at[idx_v.at[pl.ds(g * CHUNK, CHUNK)]],
            rows_v.at[b],
            sem_in[b],
        ).wait()

    def issue_write(g, b):
        pltpu.async_copy(
            rows_v.at[b],
            out_hbm.at[pl.ds(base + g * CHUNK, CHUNK)],
            sem_out[b],
        )

    def wait_write(g, b):
        pltpu.make_async_copy(
            rows_v.at[b],
            out_hbm.at[pl.ds(base + g * CHUNK, CHUNK)],
            sem_out[b],
        ).wait()

    def step(g, b, write_wait, issue):
        wait_gather(g, b)
        issue_write(g, b)
        bb = (b + LOOKAHEAD) % NBUF
        if issue:
            if write_wait:
                wait_write(g + LOOKAHEAD - NBUF, bb)
            issue_gather(g + LOOKAHEAD, bb)

    for b in range(LOOKAHEAD):
        issue_gather(b, b)

    for b in range(NBUF):
        step(b, b, write_wait=(b + LOOKAHEAD - NBUF) >= 0, issue=True)

    @pl.loop(NBUF, NCHUNK - NBUF, step=NBUF)
    def _block(g0):
        for b in range(NBUF):
            step(g0 + b, b, write_wait=True, issue=True)

    for b in range(NBUF):
        g = NCHUNK - NBUF + b
        step(g, b, write_wait=True, issue=(g + LOOKAHEAD) < NCHUNK)

    for g in range(NCHUNK - NBUF, NCHUNK):
        wait_write(g, g % NBUF)


@jax.jit
def _embed(idx_flat, table):
    mesh = plsc.VectorSubcoreMesh(core_axis_name="c", subcore_axis_name="s")
    f = pl.kernel(
        _gather_kernel,
        out_type=jax.ShapeDtypeStruct((FLAT, EMBED_DIM), jnp.float32),
        mesh=mesh,
        scratch_types=(
            [
                pltpu.VMEM((PER_W,), jnp.int32),
                pltpu.VMEM((NBUF, CHUNK, EMBED_DIM), jnp.float32),
            ]
            + [pltpu.SemaphoreType.DMA] * (2 * NBUF)
        ),
    )
    return f(idx_flat, table)


def kernel(lang_x, table):
    idx_flat = lang_x.reshape(FLAT).astype(jnp.int32)
    out = _embed(idx_flat, table)
    return out.reshape(lang_x.shape[0], lang_x.shape[1], EMBED_DIM)
